# baseline (speedup 1.0000x reference)
import jax
import jax.numpy as jnp
from jax.experimental import pallas as pl
from jax.experimental.pallas import tpu as pltpu

B, F_IN, F_OUT = 64, 4096, 4096
ROWS = 128
NC = F_OUT // ROWS
DEPTH = 8


def _mm_body(x_ref, w_hbm, m_ref, o_ref, wbuf, sems):
    i = pl.program_id(0)

    def start(c):
        slot = jax.lax.rem(c, DEPTH)
        pltpu.make_async_copy(
            w_hbm.at[pl.ds(c * ROWS, ROWS), :], wbuf.at[slot],
            sems.at[slot]).start()

    @pl.when(i == 0)
    def _prologue():
        for c in range(DEPTH):
            start(c)

    slot = jax.lax.rem(i, DEPTH)
    pltpu.make_async_copy(
        w_hbm.at[pl.ds(i * ROWS, ROWS), :], wbuf.at[slot],
        sems.at[slot]).wait()

    wm = wbuf[slot] * m_ref[...].astype(jnp.float32)
    o_ref[:, pl.ds(i * ROWS, ROWS)] = jax.lax.dot_general(
        x_ref[...], wm, (((1,), (1,)), ((), ())),
        preferred_element_type=jnp.float32)

    @pl.when(i + DEPTH < NC)
    def _refill():
        start(i + DEPTH)


def kernel(x, weight, mask):
    return pl.pallas_call(
        _mm_body,
        grid=(NC,),
        in_specs=[
            pl.BlockSpec((B, F_IN), lambda i: (0, 0)),
            pl.BlockSpec(memory_space=pltpu.MemorySpace.HBM),
            pl.BlockSpec((ROWS, F_IN), lambda i: (i, 0)),
        ],
        out_specs=pl.BlockSpec((B, F_OUT), lambda i: (0, 0)),
        out_shape=jax.ShapeDtypeStruct((B, F_OUT), jnp.float32),
        scratch_shapes=[
            pltpu.VMEM((DEPTH, ROWS, F_IN), jnp.float32),
            pltpu.SemaphoreType.DMA((DEPTH,)),
        ],
        compiler_params=pltpu.CompilerParams(
            dimension_semantics=("arbitrary",)),
    )(x, weight, mask)


# speedup vs baseline: 1.7981x; 1.7981x over previous
import jax
import jax.numpy as jnp
from jax.experimental import pallas as pl
from jax.experimental.pallas import tpu as pltpu

B, F_IN, F_OUT = 64, 4096, 4096
OB = 512


def _mm_body(x_ref, w_ref, m_ref, o_ref):
    wm = w_ref[...] * m_ref[...].astype(jnp.float32)
    o_ref[...] = jax.lax.dot_general(
        x_ref[...], wm, (((1,), (1,)), ((), ())),
        preferred_element_type=jnp.float32)


def kernel(x, weight, mask):
    m8 = mask.astype(jnp.int8)
    grid = (F_OUT // OB,)
    return pl.pallas_call(
        _mm_body,
        grid=grid,
        in_specs=[
            pl.BlockSpec((B, F_IN), lambda o: (0, 0)),
            pl.BlockSpec((OB, F_IN), lambda o: (o, 0)),
            pl.BlockSpec((OB, F_IN), lambda o: (o, 0)),
        ],
        out_specs=pl.BlockSpec((B, OB), lambda o: (0, o)),
        out_shape=jax.ShapeDtypeStruct((B, F_OUT), jnp.float32),
        compiler_params=pltpu.CompilerParams(
            dimension_semantics=("arbitrary",)),
    )(x, weight, m8)
